# baseline (speedup 1.0000x reference)
import jax
import jax.numpy as jnp
from jax.experimental import pallas as pl
from jax.experimental.pallas import tpu as pltpu

NEG_SLOPE = 0.2
F32 = jnp.float32
BF16 = jnp.bfloat16


def _proj_kernel(x_ref, w_ref, o_ref):
    o_ref[...] = jnp.dot(x_ref[...].astype(BF16), w_ref[...],
                         preferred_element_type=jnp.float32)


def _projections(x, w_all, tm):
    n, f = x.shape
    cw = w_all.shape[1]
    return pl.pallas_call(
        _proj_kernel,
        grid=(n // tm,),
        in_specs=[pl.BlockSpec((tm, f), lambda i: (i, 0)),
                  pl.BlockSpec((f, cw), lambda i: (0, 0))],
        out_specs=pl.BlockSpec((tm, cw), lambda i: (i, 0)),
        out_shape=jax.ShapeDtypeStruct((n, cw), F32),
        compiler_params=pltpu.CompilerParams(
            dimension_semantics=("parallel",)),
    )(x, w_all)


def _agg_kernel(s_ref, xp_ref, xn_ref, xr_ref, deg_ref, b_ref, o_ref):
    agg = jnp.dot(s_ref[...], xp_ref[...],
                  preferred_element_type=jnp.float32)
    srow = jnp.sum(s_ref[...].astype(F32), axis=1, keepdims=True)
    deg = deg_ref[...]
    invd = 1.0 / jnp.maximum(deg, 1.0)
    negs = (deg - srow) * invd
    xr = xr_ref[...]
    b = b_ref[...]
    cp = xp_ref.shape[1]
    left = agg * invd + xr[:, :cp] + b[:, :cp]
    right = xn_ref[...] * negs + xr[:, cp:] + b[:, cp:]
    o_ref[:, :cp] = left
    o_ref[:, cp:] = right


def _aggregate(s_mat, xp, xn, xr, deg, bias, tm):
    n = s_mat.shape[0]
    cp = xp.shape[1]
    c = xr.shape[1]
    return pl.pallas_call(
        _agg_kernel,
        grid=(n // tm,),
        in_specs=[pl.BlockSpec((tm, n), lambda i: (i, 0)),
                  pl.BlockSpec((n, cp), lambda i: (0, 0)),
                  pl.BlockSpec((tm, cp), lambda i: (i, 0)),
                  pl.BlockSpec((tm, c), lambda i: (i, 0)),
                  pl.BlockSpec((tm, 1), lambda i: (i, 0)),
                  pl.BlockSpec((1, c), lambda i: (0, 0))],
        out_specs=pl.BlockSpec((tm, c), lambda i: (i, 0)),
        out_shape=jax.ShapeDtypeStruct((n, c), F32),
        compiler_params=pltpu.CompilerParams(
            dimension_semantics=("parallel",)),
    )(s_mat, xp, xn, xr, deg, bias)


def kernel(x, edge_index, w1_t, att_l, att_r, wp_t, bp, wn_t, bn, wr_t):
    n, f = x.shape
    c = wr_t.shape[1]
    cp = wp_t.shape[1]

    x = x.astype(F32)
    w1_t = w1_t.astype(F32)
    att_l = att_l.astype(F32)
    att_r = att_r.astype(F32)

    v = jnp.dot(w1_t, jnp.concatenate([att_l, att_r], axis=0).T)
    w_all = jnp.concatenate(
        [v, wp_t.astype(F32), wn_t.astype(F32), wr_t.astype(F32)],
        axis=1).astype(BF16)
    cw = w_all.shape[1]
    pad_w = (-cw) % 128
    if pad_w:
        w_all = jnp.pad(w_all, ((0, 0), (0, pad_w)))

    xw = _projections(x, w_all, tm=512)
    sigma_l = xw[:, 0]
    sigma_r = xw[:, 1]
    xp = xw[:, 2:2 + cp].astype(BF16)
    xn = xw[:, 2 + cp:2 + 2 * cp]
    xr = xw[:, 2 + 2 * cp:2 + 2 * cp + c]

    src = edge_index[0]
    dst = edge_index[1]
    sigma_e = jax.nn.sigmoid(
        jax.nn.leaky_relu(sigma_l[src] + sigma_r[dst], NEG_SLOPE))

    s_mat = jnp.zeros((n, n), BF16).at[dst, src].add(sigma_e.astype(BF16))
    deg = jnp.zeros((n,), F32).at[dst].add(1.0).reshape(n, 1)

    bias = jnp.concatenate([bp.astype(F32), bn.astype(F32)]).reshape(1, c)

    out = _aggregate(s_mat, xp, xn, xr, deg, bias, tm=256)
    return out, sigma_e


# speedup vs baseline: 8.1324x; 8.1324x over previous
import jax
import jax.numpy as jnp
from jax.experimental import pallas as pl
from jax.experimental.pallas import tpu as pltpu

NEG_SLOPE = 0.2
F32 = jnp.float32
BF16 = jnp.bfloat16

_TM = 256
_CH = 256


def _proj_kernel(x_ref, w_ref, o_ref):
    o_ref[...] = jnp.dot(x_ref[...].astype(BF16), w_ref[...],
                         preferred_element_type=jnp.float32)


def _projections(x, w_all, tm):
    n, f = x.shape
    cw = w_all.shape[1]
    return pl.pallas_call(
        _proj_kernel,
        grid=(n // tm,),
        in_specs=[pl.BlockSpec((tm, f), lambda i: (i, 0)),
                  pl.BlockSpec((f, cw), lambda i: (0, 0))],
        out_specs=pl.BlockSpec((tm, cw), lambda i: (i, 0)),
        out_shape=jax.ShapeDtypeStruct((n, cw), F32),
        compiler_params=pltpu.CompilerParams(
            dimension_semantics=("parallel",)),
    )(x, w_all)


def _make_gate_kernel(n_rows, blk):
    def _gate_kernel(src_ref, dst_ref, sl_ref, sr_ref, o_ref):
        src = src_ref[...]
        dst = dst_ref[...]
        lo_s = jnp.bitwise_and(src, 127)
        hi_s = jnp.right_shift(src, 7)
        lo_d = jnp.bitwise_and(dst, 127)
        hi_d = jnp.right_shift(dst, 7)
        sl = jnp.zeros((blk, 128), F32)
        sr = jnp.zeros((blk, 128), F32)
        for r in range(n_rows):
            tl = jnp.broadcast_to(sl_ref[r:r + 1, :], (blk, 128))
            tr = jnp.broadcast_to(sr_ref[r:r + 1, :], (blk, 128))
            sl = jnp.where(hi_s == r, jnp.take_along_axis(tl, lo_s, axis=1), sl)
            sr = jnp.where(hi_d == r, jnp.take_along_axis(tr, lo_d, axis=1), sr)
        s = sl + sr
        s = jnp.where(s >= 0, s, NEG_SLOPE * s)
        o_ref[...] = jax.nn.sigmoid(s)
    return _gate_kernel


def _gate(src2d, dst2d, sl2d, sr2d, n_par=2):
    er, _ = src2d.shape
    n_rows = sl2d.shape[0]
    blk = er // n_par
    return pl.pallas_call(
        _make_gate_kernel(n_rows, blk),
        grid=(n_par,),
        in_specs=[pl.BlockSpec((blk, 128), lambda i: (i, 0)),
                  pl.BlockSpec((blk, 128), lambda i: (i, 0)),
                  pl.BlockSpec((n_rows, 128), lambda i: (0, 0)),
                  pl.BlockSpec((n_rows, 128), lambda i: (0, 0))],
        out_specs=pl.BlockSpec((blk, 128), lambda i: (i, 0)),
        out_shape=jax.ShapeDtypeStruct((er, 128), F32),
        compiler_params=pltpu.CompilerParams(
            dimension_semantics=("parallel",)),
    )(src2d, dst2d, sl2d, sr2d)


def _agg_kernel(rowptr_ref, srcs_ref, dsts_ref, sig_ref, xpe_ref,
                xn_ref, xr_ref, b_ref, o_ref, slot_ref):
    i = pl.program_id(0)
    base = i * _TM
    s = rowptr_ref[i]
    e = rowptr_ref[i + 1]
    j0 = s // _CH
    j1 = (e + _CH - 1) // _CH
    rows = jax.lax.broadcasted_iota(jnp.int32, (_TM, _CH), 0) + base
    cp = 64

    def chunk_body(j, carry):
        acc, deg = carry
        dstc = jnp.broadcast_to(dsts_ref[pl.ds(j, 1), :], (_TM, _CH))
        sigc = jnp.broadcast_to(sig_ref[pl.ds(j, 1), :], (_TM, _CH))
        m_plain = dstc == rows
        m_sig = jnp.where(m_plain, sigc, 0.0).astype(BF16)
        deg = deg + jnp.sum(m_plain.astype(F32), axis=1, keepdims=True)
        off = j * _CH
        for t in range(_CH):
            idx = srcs_ref[off + t]
            slot_ref[pl.ds(t, 1), :] = xpe_ref[pl.ds(idx, 1), :]
        g = slot_ref[...].astype(BF16)
        acc = acc + jnp.dot(m_sig, g, preferred_element_type=F32)
        return acc, deg

    acc0 = jnp.zeros((_TM, 128), F32)
    deg0 = jnp.zeros((_TM, 1), F32)
    acc, deg = jax.lax.fori_loop(j0, j1, chunk_body, (acc0, deg0))

    sum_s = acc[:, cp:cp + 1]
    invd = 1.0 / jnp.maximum(deg, 1.0)
    negs = (deg - sum_s) * invd
    xr = xr_ref[...]
    b = b_ref[...]
    o_ref[:, :cp] = acc[:, :cp] * invd + xr[:, :cp] + b[:, :cp]
    o_ref[:, cp:] = xn_ref[...] * negs + xr[:, cp:] + b[:, cp:]


def _aggregate(rowptr, srcs, dst2s, sig2s, xp_ext, xn, xr, bias):
    n = xp_ext.shape[0]
    nch = dst2s.shape[0]
    c = xr.shape[1]
    grid_spec = pltpu.PrefetchScalarGridSpec(
        num_scalar_prefetch=2,
        grid=(n // _TM,),
        in_specs=[
            pl.BlockSpec((nch, _CH), lambda i, *_: (0, 0)),
            pl.BlockSpec((nch, _CH), lambda i, *_: (0, 0)),
            pl.BlockSpec((n, 128), lambda i, *_: (0, 0)),
            pl.BlockSpec((_TM, 64), lambda i, *_: (i, 0)),
            pl.BlockSpec((_TM, c), lambda i, *_: (i, 0)),
            pl.BlockSpec((1, c), lambda i, *_: (0, 0)),
        ],
        out_specs=pl.BlockSpec((_TM, c), lambda i, *_: (i, 0)),
        scratch_shapes=[pltpu.VMEM((_CH, 128), F32)],
    )
    return pl.pallas_call(
        _agg_kernel,
        grid_spec=grid_spec,
        out_shape=jax.ShapeDtypeStruct((n, c), F32),
        compiler_params=pltpu.CompilerParams(
            dimension_semantics=("parallel",)),
    )(rowptr, srcs, dst2s, sig2s, xp_ext, xn, xr, bias)


def kernel(x, edge_index, w1_t, att_l, att_r, wp_t, bp, wn_t, bn, wr_t):
    n, f = x.shape
    c = wr_t.shape[1]
    cp = wp_t.shape[1]
    e = edge_index.shape[1]

    x = x.astype(F32)

    v = jnp.dot(w1_t.astype(F32),
                jnp.concatenate([att_l, att_r], axis=0).T.astype(F32))
    w_all = jnp.concatenate(
        [v, wp_t.astype(F32), wn_t.astype(F32), wr_t.astype(F32)],
        axis=1).astype(BF16)

    xw = _projections(x, w_all, tm=512)
    sl2d = xw[:, 0].reshape(n // 128, 128)
    sr2d = xw[:, 1].reshape(n // 128, 128)
    xp = xw[:, 2:2 + cp]
    xn = xw[:, 2 + cp:2 + 2 * cp]
    xr = xw[:, 2 + 2 * cp:2 + 2 * cp + c]

    src = edge_index[0].astype(jnp.int32)
    dst = edge_index[1].astype(jnp.int32)

    sig2d = _gate(src.reshape(e // 128, 128), dst.reshape(e // 128, 128),
                  sl2d, sr2d)
    sigma_e = sig2d.reshape(e)

    dst_s, src_s, sig_s = jax.lax.sort((dst, src, sigma_e), num_keys=1)

    bounds = jnp.arange(0, n + 1, _TM, dtype=jnp.int32)
    rowptr = jnp.sum(dst_s[None, :] < bounds[:, None],
                     axis=1).astype(jnp.int32)

    xp_ext = jnp.concatenate(
        [xp, jnp.ones((n, 1), F32), jnp.zeros((n, 128 - cp - 1), F32)],
        axis=1)

    bias = jnp.concatenate([bp.astype(F32), bn.astype(F32)]).reshape(1, c)

    out = _aggregate(rowptr, src_s, dst_s.reshape(e // _CH, _CH),
                     sig_s.reshape(e // _CH, _CH), xp_ext, xn, xr, bias)
    return out, sigma_e


# speedup vs baseline: 9.2220x; 1.1340x over previous
import jax
import jax.numpy as jnp
from jax.experimental import pallas as pl
from jax.experimental.pallas import tpu as pltpu

NEG_SLOPE = 0.2
F32 = jnp.float32
BF16 = jnp.bfloat16

_TM = 256
_CH = 256


def _proj_kernel(x_ref, w_ref, o_ref):
    o_ref[...] = jnp.dot(x_ref[...].astype(BF16), w_ref[...],
                         preferred_element_type=jnp.float32)


def _projections(x, w_all, tm):
    n, f = x.shape
    cw = w_all.shape[1]
    return pl.pallas_call(
        _proj_kernel,
        grid=(n // tm,),
        in_specs=[pl.BlockSpec((tm, f), lambda i: (i, 0)),
                  pl.BlockSpec((f, cw), lambda i: (0, 0))],
        out_specs=pl.BlockSpec((tm, cw), lambda i: (i, 0)),
        out_shape=jax.ShapeDtypeStruct((n, cw), F32),
        compiler_params=pltpu.CompilerParams(
            dimension_semantics=("parallel",)),
    )(x, w_all)


def _make_gate_kernel(n_rows, blk):
    def _gate_kernel(src_ref, dst_ref, sl_ref, sr_ref, o_ref):
        src = src_ref[...]
        dst = dst_ref[...]
        lo_s = jnp.bitwise_and(src, 127)
        hi_s = jnp.right_shift(src, 7)
        lo_d = jnp.bitwise_and(dst, 127)
        hi_d = jnp.right_shift(dst, 7)
        sl = jnp.zeros((blk, 128), F32)
        sr = jnp.zeros((blk, 128), F32)
        for r in range(n_rows):
            tl = jnp.broadcast_to(sl_ref[r:r + 1, :], (blk, 128))
            tr = jnp.broadcast_to(sr_ref[r:r + 1, :], (blk, 128))
            sl = jnp.where(hi_s == r, jnp.take_along_axis(tl, lo_s, axis=1), sl)
            sr = jnp.where(hi_d == r, jnp.take_along_axis(tr, lo_d, axis=1), sr)
        s = sl + sr
        s = jnp.where(s >= 0, s, NEG_SLOPE * s)
        o_ref[...] = jax.nn.sigmoid(s)
    return _gate_kernel


def _gate(src2d, dst2d, sl2d, sr2d, n_par=2):
    er, _ = src2d.shape
    n_rows = sl2d.shape[0]
    blk = er // n_par
    return pl.pallas_call(
        _make_gate_kernel(n_rows, blk),
        grid=(n_par,),
        in_specs=[pl.BlockSpec((blk, 128), lambda i: (i, 0)),
                  pl.BlockSpec((blk, 128), lambda i: (i, 0)),
                  pl.BlockSpec((n_rows, 128), lambda i: (0, 0)),
                  pl.BlockSpec((n_rows, 128), lambda i: (0, 0))],
        out_specs=pl.BlockSpec((blk, 128), lambda i: (i, 0)),
        out_shape=jax.ShapeDtypeStruct((er, 128), F32),
        compiler_params=pltpu.CompilerParams(
            dimension_semantics=("parallel",)),
    )(src2d, dst2d, sl2d, sr2d)


def _agg_kernel(rowptr_ref, srcs_ref, dsts_ref, sig_ref, xpe_ref,
                xn_ref, xr_ref, b_ref, o_ref, slot0_ref, slot1_ref):
    i = pl.program_id(0)
    base = i * _TM
    s = rowptr_ref[i]
    e = rowptr_ref[i + 1]
    jp0 = (s // _CH) // 2
    jp1 = ((e + _CH - 1) // _CH + 1) // 2
    rows = jax.lax.broadcasted_iota(jnp.int32, (_TM, _CH), 0) + base
    cp = 64

    def one_chunk(j, slot_ref, acc, deg):
        dstc = jnp.broadcast_to(dsts_ref[pl.ds(j, 1), :], (_TM, _CH))
        sigc = jnp.broadcast_to(sig_ref[pl.ds(j, 1), :], (_TM, _CH))
        m_plain = dstc == rows
        m_sig = jnp.where(m_plain, sigc, 0.0).astype(BF16)
        deg = deg + jnp.sum(m_plain.astype(F32), axis=1, keepdims=True)
        off = j * _CH
        for t in range(_CH):
            idx = srcs_ref[off + t]
            slot_ref[pl.ds(t, 1), :] = xpe_ref[pl.ds(idx, 1), :]
        g = slot_ref[...].astype(BF16)
        acc = acc + jnp.dot(m_sig, g, preferred_element_type=F32)
        return acc, deg

    def pair_body(jp, carry):
        acc, deg = carry
        acc, deg = one_chunk(2 * jp, slot0_ref, acc, deg)
        acc, deg = one_chunk(2 * jp + 1, slot1_ref, acc, deg)
        return acc, deg

    acc0 = jnp.zeros((_TM, 128), F32)
    deg0 = jnp.zeros((_TM, 1), F32)
    acc, deg = jax.lax.fori_loop(jp0, jp1, pair_body, (acc0, deg0))

    sum_s = acc[:, cp:cp + 1]
    invd = 1.0 / jnp.maximum(deg, 1.0)
    negs = (deg - sum_s) * invd
    xr = xr_ref[...]
    b = b_ref[...]
    o_ref[:, :cp] = acc[:, :cp] * invd + xr[:, :cp] + b[:, :cp]
    o_ref[:, cp:] = xn_ref[...] * negs + xr[:, cp:] + b[:, cp:]


def _aggregate(rowptr, srcs, dst2s, sig2s, xp_ext, xn, xr, bias):
    n = xp_ext.shape[0]
    nch = dst2s.shape[0]
    c = xr.shape[1]
    grid_spec = pltpu.PrefetchScalarGridSpec(
        num_scalar_prefetch=2,
        grid=(n // _TM,),
        in_specs=[
            pl.BlockSpec((nch, _CH), lambda i, *_: (0, 0)),
            pl.BlockSpec((nch, _CH), lambda i, *_: (0, 0)),
            pl.BlockSpec((n, 128), lambda i, *_: (0, 0)),
            pl.BlockSpec((_TM, 64), lambda i, *_: (i, 0)),
            pl.BlockSpec((_TM, c), lambda i, *_: (i, 0)),
            pl.BlockSpec((1, c), lambda i, *_: (0, 0)),
        ],
        out_specs=pl.BlockSpec((_TM, c), lambda i, *_: (i, 0)),
        scratch_shapes=[pltpu.VMEM((_CH, 128), F32),
                        pltpu.VMEM((_CH, 128), F32)],
    )
    return pl.pallas_call(
        _agg_kernel,
        grid_spec=grid_spec,
        out_shape=jax.ShapeDtypeStruct((n, c), F32),
        compiler_params=pltpu.CompilerParams(
            dimension_semantics=("parallel",)),
    )(rowptr, srcs, dst2s, sig2s, xp_ext, xn, xr, bias)


def kernel(x, edge_index, w1_t, att_l, att_r, wp_t, bp, wn_t, bn, wr_t):
    n, f = x.shape
    c = wr_t.shape[1]
    cp = wp_t.shape[1]
    e = edge_index.shape[1]

    x = x.astype(F32)

    v = jnp.dot(w1_t.astype(F32),
                jnp.concatenate([att_l, att_r], axis=0).T.astype(F32))
    w_all = jnp.concatenate(
        [v, wp_t.astype(F32), wn_t.astype(F32), wr_t.astype(F32)],
        axis=1).astype(BF16)

    xw = _projections(x, w_all, tm=512)
    sl2d = xw[:, 0].reshape(n // 128, 128)
    sr2d = xw[:, 1].reshape(n // 128, 128)
    xp = xw[:, 2:2 + cp]
    xn = xw[:, 2 + cp:2 + 2 * cp]
    xr = xw[:, 2 + 2 * cp:2 + 2 * cp + c]

    src = edge_index[0].astype(jnp.int32)
    dst = edge_index[1].astype(jnp.int32)

    sig2d = _gate(src.reshape(e // 128, 128), dst.reshape(e // 128, 128),
                  sl2d, sr2d)
    sigma_e = sig2d.reshape(e)

    key = dst * 4096 + src
    key_s, sig_s = jax.lax.sort((key, sigma_e), num_keys=1)
    dst_s = jnp.right_shift(key_s, 12)
    src_s = jnp.bitwise_and(key_s, 4095)

    bounds = jnp.arange(0, n + 1, _TM, dtype=jnp.int32)
    rowptr = jnp.sum(dst_s[None, :] < bounds[:, None],
                     axis=1).astype(jnp.int32)

    xp_ext = jnp.concatenate(
        [xp, jnp.ones((n, 1), F32), jnp.zeros((n, 128 - cp - 1), F32)],
        axis=1)

    bias = jnp.concatenate([bp.astype(F32), bn.astype(F32)]).reshape(1, c)

    out = _aggregate(rowptr, src_s, dst_s.reshape(e // _CH, _CH),
                     sig_s.reshape(e // _CH, _CH), xp_ext, xn, xr, bias)
    return out, sigma_e


# speedup vs baseline: 9.9443x; 1.0783x over previous
import jax
import jax.numpy as jnp
from jax.experimental import pallas as pl
from jax.experimental.pallas import tpu as pltpu

NEG_SLOPE = 0.2
F32 = jnp.float32
BF16 = jnp.bfloat16

_TM = 256
_CH = 256


def _make_proj_kernel(tm, cp, c):
    def _proj_kernel(x_ref, w_ref, sl_ref, sr_ref, xpe_ref, xn_ref, xr_ref):
        y = jnp.dot(x_ref[...].astype(BF16), w_ref[...],
                    preferred_element_type=jnp.float32)
        sl_ref[...] = y[:, 0].reshape(1, tm // 128, 128)
        sr_ref[...] = y[:, 1].reshape(1, tm // 128, 128)
        xpe_ref[:, :cp] = y[:, 2:2 + cp]
        xpe_ref[:, cp:cp + 1] = jnp.ones((tm, 1), F32)
        xpe_ref[:, cp + 1:] = jnp.zeros((tm, 128 - cp - 1), F32)
        xn_ref[...] = y[:, 2 + cp:2 + 2 * cp]
        xr_ref[...] = y[:, 2 + 2 * cp:2 + 2 * cp + c]
    return _proj_kernel


def _projections(x, w_all, cp, c, tm):
    n, f = x.shape
    cw = w_all.shape[1]
    tr = tm // 128
    return pl.pallas_call(
        _make_proj_kernel(tm, cp, c),
        grid=(n // tm,),
        in_specs=[pl.BlockSpec((tm, f), lambda i: (i, 0)),
                  pl.BlockSpec((f, cw), lambda i: (0, 0))],
        out_specs=[pl.BlockSpec((1, tr, 128), lambda i: (i, 0, 0)),
                   pl.BlockSpec((1, tr, 128), lambda i: (i, 0, 0)),
                   pl.BlockSpec((tm, 128), lambda i: (i, 0)),
                   pl.BlockSpec((tm, cp), lambda i: (i, 0)),
                   pl.BlockSpec((tm, c), lambda i: (i, 0))],
        out_shape=[jax.ShapeDtypeStruct((n // tm, tr, 128), F32),
                   jax.ShapeDtypeStruct((n // tm, tr, 128), F32),
                   jax.ShapeDtypeStruct((n, 128), F32),
                   jax.ShapeDtypeStruct((n, cp), F32),
                   jax.ShapeDtypeStruct((n, c), F32)],
        compiler_params=pltpu.CompilerParams(
            dimension_semantics=("parallel",)),
    )(x, w_all)


def _tree_sum(vs):
    while len(vs) > 1:
        vs = [a + b for a, b in zip(vs[::2], vs[1::2])] + (
            [vs[-1]] if len(vs) % 2 else [])
    return vs[0]


def _make_gate_kernel(n_rows, blk):
    def _gate_kernel(src_ref, dst_ref, sl_ref, sr_ref, o_ref):
        src = src_ref[...]
        dst = dst_ref[...]
        lo_s = jnp.bitwise_and(src, 127)
        hi_s = jnp.right_shift(src, 7)
        lo_d = jnp.bitwise_and(dst, 127)
        hi_d = jnp.right_shift(dst, 7)
        parts_l, parts_r = [], []
        for r in range(n_rows):
            tl = jnp.broadcast_to(sl_ref[r:r + 1, :], (blk, 128))
            tr = jnp.broadcast_to(sr_ref[r:r + 1, :], (blk, 128))
            parts_l.append(jnp.where(
                hi_s == r, jnp.take_along_axis(tl, lo_s, axis=1), 0.0))
            parts_r.append(jnp.where(
                hi_d == r, jnp.take_along_axis(tr, lo_d, axis=1), 0.0))
        s = _tree_sum(parts_l) + _tree_sum(parts_r)
        s = jnp.where(s >= 0, s, NEG_SLOPE * s)
        o_ref[...] = jax.nn.sigmoid(s)
    return _gate_kernel


def _gate(src2d, dst2d, sl2d, sr2d, n_par=2):
    er, _ = src2d.shape
    n_rows = sl2d.shape[0]
    blk = er // n_par
    return pl.pallas_call(
        _make_gate_kernel(n_rows, blk),
        grid=(n_par,),
        in_specs=[pl.BlockSpec((blk, 128), lambda i: (i, 0)),
                  pl.BlockSpec((blk, 128), lambda i: (i, 0)),
                  pl.BlockSpec((n_rows, 128), lambda i: (0, 0)),
                  pl.BlockSpec((n_rows, 128), lambda i: (0, 0))],
        out_specs=pl.BlockSpec((blk, 128), lambda i: (i, 0)),
        out_shape=jax.ShapeDtypeStruct((er, 128), F32),
        compiler_params=pltpu.CompilerParams(
            dimension_semantics=("parallel",)),
    )(src2d, dst2d, sl2d, sr2d)


def _agg_kernel(rowptr_ref, srcs_ref, keys_ref, sl_ref, sr_ref, xpe_ref,
                xn_ref, xr_ref, b_ref, o_ref, slot0_ref, slot1_ref):
    i = pl.program_id(0)
    base = i * _TM
    s = rowptr_ref[i]
    e = rowptr_ref[i + 1]
    jp0 = (s // _CH) // 2
    jp1 = ((e + _CH - 1) // _CH + 1) // 2
    rows = jax.lax.broadcasted_iota(jnp.int32, (_TM, _CH), 0) + base
    cp = 64
    n_rows = sl_ref.shape[0]

    def lookup(tbl_ref, lo, hi, rs):
        vs = []
        for r in rs:
            t = jnp.broadcast_to(tbl_ref[pl.ds(r, 1), :], (2, 128))
            vs.append(jnp.where(
                hi == r, jnp.take_along_axis(t, lo, axis=1), 0.0))
        return _tree_sum(vs)

    def one_chunk(j, slot_ref, acc, deg):
        keyc = keys_ref[pl.ds(j, 1), :]
        dstc = jnp.right_shift(keyc, 12)
        k2 = keyc.reshape(2, 128)
        src2 = jnp.bitwise_and(k2, 4095)
        dst2 = jnp.right_shift(k2, 12)
        slv = lookup(sl_ref, jnp.bitwise_and(src2, 127),
                     jnp.right_shift(src2, 7), range(n_rows))
        srv = lookup(sr_ref, jnp.bitwise_and(dst2, 127),
                     jnp.right_shift(dst2, 7), [2 * i, 2 * i + 1])
        g8 = slv + srv
        g8 = jnp.where(g8 >= 0, g8, NEG_SLOPE * g8)
        sigc = jax.nn.sigmoid(g8).reshape(1, _CH)

        m_plain = jnp.broadcast_to(dstc, (_TM, _CH)) == rows
        m_sig = jnp.where(m_plain, jnp.broadcast_to(sigc, (_TM, _CH)),
                          0.0).astype(BF16)
        deg = deg + jnp.sum(m_plain.astype(F32), axis=1, keepdims=True)
        off = j * _CH
        for t0 in range(0, _CH, 8):
            vals = [xpe_ref[pl.ds(srcs_ref[off + t0 + u], 1), :]
                    for u in range(8)]
            for u in range(8):
                slot_ref[pl.ds(t0 + u, 1), :] = vals[u]
        g = slot_ref[...].astype(BF16)
        acc = acc + jnp.dot(m_sig, g, preferred_element_type=F32)
        return acc, deg

    def pair_body(jp, carry):
        acc, deg = carry
        acc, deg = one_chunk(2 * jp, slot0_ref, acc, deg)
        acc, deg = one_chunk(2 * jp + 1, slot1_ref, acc, deg)
        return acc, deg

    acc0 = jnp.zeros((_TM, 128), F32)
    deg0 = jnp.zeros((_TM, 1), F32)
    acc, deg = jax.lax.fori_loop(jp0, jp1, pair_body, (acc0, deg0))

    sum_s = acc[:, cp:cp + 1]
    invd = 1.0 / jnp.maximum(deg, 1.0)
    negs = (deg - sum_s) * invd
    xr = xr_ref[...]
    b = b_ref[...]
    o_ref[:, :cp] = acc[:, :cp] * invd + xr[:, :cp] + b[:, :cp]
    o_ref[:, cp:] = xn_ref[...] * negs + xr[:, cp:] + b[:, cp:]


def _aggregate(rowptr, srcs, key2s, sl2d, sr2d, xp_ext, xn, xr, bias):
    n = xp_ext.shape[0]
    nch = key2s.shape[0]
    nr = sl2d.shape[0]
    c = xr.shape[1]
    grid_spec = pltpu.PrefetchScalarGridSpec(
        num_scalar_prefetch=2,
        grid=(n // _TM,),
        in_specs=[
            pl.BlockSpec((nch, _CH), lambda i, *_: (0, 0)),
            pl.BlockSpec((nr, 128), lambda i, *_: (0, 0)),
            pl.BlockSpec((nr, 128), lambda i, *_: (0, 0)),
            pl.BlockSpec((n, 128), lambda i, *_: (0, 0)),
            pl.BlockSpec((_TM, 64), lambda i, *_: (i, 0)),
            pl.BlockSpec((_TM, c), lambda i, *_: (i, 0)),
            pl.BlockSpec((1, c), lambda i, *_: (0, 0)),
        ],
        out_specs=pl.BlockSpec((_TM, c), lambda i, *_: (i, 0)),
        scratch_shapes=[pltpu.VMEM((_CH, 128), F32),
                        pltpu.VMEM((_CH, 128), F32)],
    )
    return pl.pallas_call(
        _agg_kernel,
        grid_spec=grid_spec,
        out_shape=jax.ShapeDtypeStruct((n, c), F32),
        compiler_params=pltpu.CompilerParams(
            dimension_semantics=("parallel",)),
    )(rowptr, srcs, key2s, sl2d, sr2d, xp_ext, xn, xr, bias)


def kernel(x, edge_index, w1_t, att_l, att_r, wp_t, bp, wn_t, bn, wr_t):
    n, f = x.shape
    c = wr_t.shape[1]
    cp = wp_t.shape[1]
    e = edge_index.shape[1]

    x = x.astype(F32)

    v = jnp.dot(w1_t.astype(F32),
                jnp.concatenate([att_l, att_r], axis=0).T.astype(F32))
    w_all = jnp.concatenate(
        [v, wp_t.astype(F32), wn_t.astype(F32), wr_t.astype(F32)],
        axis=1).astype(BF16)

    sl3d, sr3d, xp_ext, xn, xr = _projections(x, w_all, cp, c, tm=512)
    sl2d = sl3d.reshape(n // 128, 128)
    sr2d = sr3d.reshape(n // 128, 128)

    src = edge_index[0].astype(jnp.int32)
    dst = edge_index[1].astype(jnp.int32)

    sig2d = _gate(src.reshape(e // 128, 128), dst.reshape(e // 128, 128),
                  sl2d, sr2d)
    sigma_e = sig2d.reshape(e)

    key = dst * 4096 + src
    key_s = jax.lax.sort(key)
    src_s = jnp.bitwise_and(key_s, 4095)
    dst_s = jnp.right_shift(key_s, 12)

    bounds = jnp.arange(0, n + 1, _TM, dtype=jnp.int32)
    rowptr = jnp.sum(dst_s[None, :] < bounds[:, None],
                     axis=1).astype(jnp.int32)

    bias = jnp.concatenate([bp.astype(F32), bn.astype(F32)]).reshape(1, c)

    out = _aggregate(rowptr, src_s, key_s.reshape(e // _CH, _CH),
                     sl2d, sr2d, xp_ext, xn, xr, bias)
    return out, sigma_e
